# baseline (speedup 1.0000x reference)
import functools

import jax
import jax.numpy as jnp
from jax import lax
from jax.experimental import pallas as pl
from jax.experimental.pallas import tpu as pltpu
from jax.experimental.pallas import tpu_sc as plsc

_EMBED = 64
_NC = 2
_NS = 16
_NW = _NC * _NS
_ROWS = 1
_NBUF = 8


def _make_gather(batch, hist):
    rows_per_w = batch // _NW
    nchunks = rows_per_w // _ROWS
    mesh = plsc.VectorSubcoreMesh(core_axis_name="c", subcore_axis_name="s")

    @functools.partial(
        pl.kernel,
        out_type=jax.ShapeDtypeStruct((batch, hist, _EMBED), jnp.float32),
        mesh=mesh,
        scratch_types=[
            pltpu.VMEM((_NBUF, hist), jnp.int32),
            pltpu.VMEM((_NBUF, hist, _EMBED), jnp.float32),
            pltpu.SemaphoreType.DMA,
        ],
        compiler_params=pltpu.CompilerParams(use_tc_tiling_on_sc=False),
    )
    def gather(idx_hbm, table_hbm, out_hbm, idx_v, rows_v, gsem):
        wid = lax.axis_index("s") * _NC + lax.axis_index("c")
        base = wid * rows_per_w

        def load_and_fire(c, b):
            r = base + c * _ROWS
            pltpu.sync_copy(idx_hbm.at[r], idx_v.at[b])
            pltpu.make_async_copy(
                table_hbm.at[idx_v.at[b]], rows_v.at[b], gsem).start()

        def drain_and_store(c, b):
            pltpu.make_async_copy(
                table_hbm.at[idx_v.at[b]], rows_v.at[b], gsem).wait()
            r = base + c * _ROWS
            pltpu.sync_copy(rows_v.at[b], out_hbm.at[r])

        for b in range(_NBUF):
            load_and_fire(b, b)

        @pl.loop(0, nchunks - _NBUF, step=_NBUF)
        def _step(c0):
            for b in range(_NBUF):
                c = c0 + b
                drain_and_store(c, b)
                load_and_fire(c + _NBUF, b)

        for b in range(_NBUF):
            drain_and_store(nchunks - _NBUF + b, b)

    return gather


def kernel(x, vectors):
    b, h = x.shape
    return _make_gather(b, h)(x, vectors)
